# Initial kernel scaffold; baseline (speedup 1.0000x reference)
import jax
import jax.numpy as jnp
from jax.experimental import pallas as pl


def kernel(x_node, x_instance, x_svc, ei_svc_call, ei_instance_node, ei_node_instance, ei_instance_instance, ei_svc_instance, ei_instance_svc, W_conv, b_conv, W_ih, W_hh, b_ih, b_hh):
    raise NotImplementedError("write your pallas kernel here")


# speedup vs baseline: 3.6212x; 3.6212x over previous
import functools

import jax
import jax.numpy as jnp
from jax.experimental import pallas as pl
from jax.experimental.pallas import tpu as pltpu

N = 4000
E = 64000
D = 128
H = 128
T = 3 * N
TB = 600
NB = T // TB

_INTERPRET = False


def _matmul_kernel(x_ref, cnt_ref, w_ref, q_ref):
    c = cnt_ref[0, :, 0]
    s = jax.lax.rsqrt(jnp.maximum(c, 1.0))
    x = x_ref[0] * s[:, None]
    q_ref[0] = jax.lax.dot_general(
        x, w_ref[0], (((1,), (0,)), ((), ())),
        preferred_element_type=jnp.float32)


def _scaled_matmuls(src_stack, counts, W_conv):
    return pl.pallas_call(
        _matmul_kernel,
        grid=(6,),
        in_specs=[
            pl.BlockSpec((1, N, D), lambda r: (r, 0, 0)),
            pl.BlockSpec((1, N, 16), lambda r: (2 * r, 0, 0)),
            pl.BlockSpec((1, D, H), lambda r: (r, 0, 0)),
        ],
        out_specs=pl.BlockSpec((1, N, H), lambda r: (r, 0, 0)),
        out_shape=jax.ShapeDtypeStruct((6, N, H), jnp.float32),
        interpret=_INTERPRET,
    )(src_stack, counts, W_conv)


def _combine_kernel(agg_ref, cnt_ref, b_ref, feat_ref):
    t = pl.program_id(0)
    inv_m = jnp.where(t == 0, 1.0, jnp.where(t == 1, 1.0 / 3.0, 0.5))
    acc = jnp.zeros((N, H), jnp.float32)
    for j in range(3):
        c = cnt_ref[0, j, :, 0]
        s = jax.lax.rsqrt(jnp.maximum(c, 1.0))
        acc = acc + agg_ref[0, j] * s[:, None]
    beff = (b_ref[0, 0] + b_ref[0, 1] + b_ref[0, 2]) * inv_m
    feat_ref[0] = jnp.maximum(acc * inv_m + beff[None, :], 0.0)


def _combine(agg_stack, cnt_stack, b_stack):
    return pl.pallas_call(
        _combine_kernel,
        grid=(3,),
        in_specs=[
            pl.BlockSpec((1, 3, N, H), lambda t: (t, 0, 0, 0)),
            pl.BlockSpec((1, 3, N, 16), lambda t: (t, 0, 0, 0)),
            pl.BlockSpec((1, 3, H), lambda t: (t, 0, 0)),
        ],
        out_specs=pl.BlockSpec((1, N, H), lambda t: (t, 0, 0)),
        out_shape=jax.ShapeDtypeStruct((3, N, H), jnp.float32),
        interpret=_INTERPRET,
    )(agg_stack, cnt_stack, b_stack)


def _lstm_step(g, c):
    i = jax.nn.sigmoid(g[:, 0:H])
    f = jax.nn.sigmoid(g[:, H:2 * H])
    gg = jnp.tanh(g[:, 2 * H:3 * H])
    o = jax.nn.sigmoid(g[:, 3 * H:4 * H])
    c = f * c + i * gg
    h = o * jnp.tanh(c)
    return h, c


def _lstm_kernel(feat_ref, wih_ref, whh_ref, b_ref, out_ref,
                 xp_scr, h1blk, carry):
    k = pl.program_id(0)

    @pl.when(k == 0)
    def _init():
        carry[...] = jnp.zeros_like(carry)

    mm = functools.partial(
        jax.lax.dot_general, dimension_numbers=(((1,), (0,)), ((), ())),
        preferred_element_type=jnp.float32)

    b1 = b_ref[0] + b_ref[1]
    b2 = b_ref[2] + b_ref[3]
    whh0 = whh_ref[0]
    whh1 = whh_ref[1]

    xp_scr[...] = mm(feat_ref[...], wih_ref[0]) + b1[None, :]

    def step1(t, hc):
        h, c = hc
        g = xp_scr[pl.ds(t, 1), :] + mm(h, whh0)
        h, c = _lstm_step(g, c)
        h1blk[pl.ds(t, 1), :] = h
        return (h, c)

    h1, c1 = jax.lax.fori_loop(
        0, TB, step1, (carry[0:1, :], carry[1:2, :]))

    xp_scr[...] = mm(h1blk[...], wih_ref[1]) + b2[None, :]

    def step2(t, hc):
        h, c = hc
        g = xp_scr[pl.ds(t, 1), :] + mm(h, whh1)
        h, c = _lstm_step(g, c)
        out_ref[pl.ds(t, 1), :] = jnp.maximum(h, 0.0)
        return (h, c)

    h2, c2 = jax.lax.fori_loop(
        0, TB, step2, (carry[2:3, :], carry[3:4, :]))

    carry[0:1, :] = h1
    carry[1:2, :] = c1
    carry[2:3, :] = h2
    carry[3:4, :] = c2


def _lstm(feat, wih_t, whh_t, b_cat):
    return pl.pallas_call(
        _lstm_kernel,
        grid=(NB,),
        in_specs=[
            pl.BlockSpec((TB, D), lambda k: (k, 0)),
            pl.BlockSpec((2, D, 4 * H), lambda k: (0, 0, 0)),
            pl.BlockSpec((2, H, 4 * H), lambda k: (0, 0, 0)),
            pl.BlockSpec((4, 4 * H), lambda k: (0, 0)),
        ],
        out_specs=pl.BlockSpec((TB, H), lambda k: (k, 0)),
        out_shape=jax.ShapeDtypeStruct((T, H), jnp.float32),
        scratch_shapes=[
            pltpu.VMEM((TB, 4 * H), jnp.float32),
            pltpu.VMEM((TB, H), jnp.float32),
            pltpu.VMEM((4, H), jnp.float32),
        ],
        compiler_params=pltpu.CompilerParams(
            dimension_semantics=("arbitrary",)),
        interpret=_INTERPRET,
    )(feat, wih_t, whh_t, b_cat)


def _bincounts(idx_stack):
    def one(ix):
        c = jnp.zeros((N,), jnp.float32).at[ix].add(1.0)
        return jnp.pad(c[:, None], ((0, 0), (0, 15)))
    return jax.vmap(one)(idx_stack)


def _edge_aggregate(q_stack, src_stack, dst_stack):
    def one(q, s, d):
        return jnp.zeros((N, H), jnp.float32).at[d].add(q[s])
    return jax.vmap(one)(q_stack, src_stack, dst_stack)


def kernel(x_node, x_instance, x_svc, ei_svc_call, ei_instance_node,
           ei_node_instance, ei_instance_instance, ei_svc_instance,
           ei_instance_svc, W_conv, b_conv, W_ih, W_hh, b_ih, b_hh):
    eis = [ei_svc_call, ei_instance_node, ei_node_instance,
           ei_instance_instance, ei_svc_instance, ei_instance_svc]
    idx_stack = jnp.stack([e[i] for e in eis for i in (0, 1)])
    counts = _bincounts(idx_stack)

    src_tables = jnp.stack(
        [x_svc, x_instance, x_node, x_instance, x_svc, x_instance])
    q_stack = _scaled_matmuls(src_tables, counts, W_conv)

    src_stack = jnp.stack([e[0] for e in eis])
    dst_stack = jnp.stack([e[1] for e in eis])
    agg = _edge_aggregate(q_stack, src_stack, dst_stack)

    zeros_nh = jnp.zeros((N, H), jnp.float32)
    agg_stack = jnp.stack([
        jnp.stack([agg[1], zeros_nh, zeros_nh]),
        jnp.stack([agg[2], agg[3], agg[4]]),
        jnp.stack([agg[0], agg[5], zeros_nh]),
    ])
    ones_cnt = jnp.ones((N, 16), jnp.float32)
    cnt_stack = jnp.stack([
        jnp.stack([counts[3], ones_cnt, ones_cnt]),
        jnp.stack([counts[5], counts[7], counts[9]]),
        jnp.stack([counts[1], counts[11], ones_cnt]),
    ])
    zeros_b = jnp.zeros((H,), jnp.float32)
    b_stack = jnp.stack([
        jnp.stack([b_conv[1], zeros_b, zeros_b]),
        jnp.stack([b_conv[2], b_conv[3], b_conv[4]]),
        jnp.stack([b_conv[0], b_conv[5], zeros_b]),
    ])

    feat3 = _combine(agg_stack, cnt_stack, b_stack)
    feat = feat3.reshape(T, D)

    wih_t = jnp.transpose(W_ih, (0, 2, 1))
    whh_t = jnp.transpose(W_hh, (0, 2, 1))
    b_cat = jnp.stack([b_ih[0], b_hh[0], b_ih[1], b_hh[1]])
    rnn_out = _lstm(feat, wih_t, whh_t, b_cat)

    return (feat[:N], feat[N:2 * N], feat[2 * N:], rnn_out)


# speedup vs baseline: 4.2117x; 1.1631x over previous
import functools

import jax
import jax.numpy as jnp
from jax.experimental import pallas as pl
from jax.experimental.pallas import tpu as pltpu

N = 4000
E = 64000
D = 128
H = 128
T = 3 * N
TB = 600
NB = T // TB

_INTERPRET = False


def _matmul_kernel(x_ref, cnt_ref, w_ref, q_ref):
    c = cnt_ref[0, :, 0]
    s = jax.lax.rsqrt(jnp.maximum(c, 1.0))
    x = x_ref[0] * s[:, None]
    q_ref[0] = jax.lax.dot_general(
        x, w_ref[0], (((1,), (0,)), ((), ())),
        preferred_element_type=jnp.float32)


def _scaled_matmuls(src_stack, counts, W_conv):
    return pl.pallas_call(
        _matmul_kernel,
        grid=(6,),
        in_specs=[
            pl.BlockSpec((1, N, D), lambda r: (r, 0, 0)),
            pl.BlockSpec((1, N, 16), lambda r: (2 * r, 0, 0)),
            pl.BlockSpec((1, D, H), lambda r: (r, 0, 0)),
        ],
        out_specs=pl.BlockSpec((1, N, H), lambda r: (r, 0, 0)),
        out_shape=jax.ShapeDtypeStruct((6, N, H), jnp.float32),
        interpret=_INTERPRET,
    )(src_stack, counts, W_conv)


def _combine_kernel(agg_ref, cnt_ref, b_ref, feat_ref):
    t = pl.program_id(0)
    inv_m = jnp.where(t == 0, 1.0, jnp.where(t == 1, 1.0 / 3.0, 0.5))
    acc = jnp.zeros((N, H), jnp.float32)
    for j in range(3):
        c = cnt_ref[0, j, :, 0]
        s = jax.lax.rsqrt(jnp.maximum(c, 1.0))
        acc = acc + agg_ref[0, j] * s[:, None]
    beff = (b_ref[0, 0] + b_ref[0, 1] + b_ref[0, 2]) * inv_m
    feat_ref[0] = jnp.maximum(acc * inv_m + beff[None, :], 0.0)


def _combine(agg_stack, cnt_stack, b_stack):
    return pl.pallas_call(
        _combine_kernel,
        grid=(3,),
        in_specs=[
            pl.BlockSpec((1, 3, N, H), lambda t: (t, 0, 0, 0)),
            pl.BlockSpec((1, 3, N, 16), lambda t: (t, 0, 0, 0)),
            pl.BlockSpec((1, 3, H), lambda t: (t, 0, 0)),
        ],
        out_specs=pl.BlockSpec((1, N, H), lambda t: (t, 0, 0)),
        out_shape=jax.ShapeDtypeStruct((3, N, H), jnp.float32),
        interpret=_INTERPRET,
    )(agg_stack, cnt_stack, b_stack)


def _lstm_kernel(feat_ref, wihbd_ref, whhbd_ref, b_ref, out_ref,
                 xp_scr, h1blk, carry):
    k = pl.program_id(0)

    @pl.when(k == 0)
    def _init():
        carry[...] = jnp.zeros_like(carry)
        h1blk[...] = jnp.zeros_like(h1blk)

    @pl.when(k == 1)
    def _reset_l2():
        carry[:, H:2 * H] = jnp.zeros((2, H), jnp.float32)

    mm = functools.partial(
        jax.lax.dot_general, dimension_numbers=(((1,), (0,)), ((), ())),
        preferred_element_type=jnp.float32)

    xin = jnp.concatenate([feat_ref[...], h1blk[...]], axis=1)
    xp_scr[...] = mm(xin, wihbd_ref[...]) + b_ref[...]

    whh = whhbd_ref[...]
    h12 = carry[0:1, :]
    c12 = carry[1:2, :]

    def group(gi, hc):
        h12, c12 = hc
        base = pl.multiple_of(gi * 8, 8)
        rows = xp_scr[pl.ds(base, 8), :]
        h1s, outs = [], []
        for j in range(8):
            g = rows[j:j + 1, :] + mm(h12, whh)
            ii = jax.nn.sigmoid(g[:, 0:2 * H])
            ff = jax.nn.sigmoid(g[:, 2 * H:4 * H])
            gg = jnp.tanh(g[:, 4 * H:6 * H])
            oo = jax.nn.sigmoid(g[:, 6 * H:8 * H])
            c12 = ff * c12 + ii * gg
            h12 = oo * jnp.tanh(c12)
            h1s.append(h12[:, 0:H])
            outs.append(h12[:, H:2 * H])
        h1blk[pl.ds(base, 8), :] = jnp.concatenate(h1s, axis=0)
        out_ref[pl.ds(base, 8), :] = jnp.maximum(
            jnp.concatenate(outs, axis=0), 0.0)
        return (h12, c12)

    h12, c12 = jax.lax.fori_loop(0, TB // 8, group, (h12, c12))
    carry[0:1, :] = h12
    carry[1:2, :] = c12


def _lstm(feat, wih_bd, whh_bd, bias):
    return pl.pallas_call(
        _lstm_kernel,
        grid=(NB + 1,),
        in_specs=[
            pl.BlockSpec((TB, D), lambda k: (jnp.minimum(k, NB - 1), 0)),
            pl.BlockSpec((2 * H, 8 * H), lambda k: (0, 0)),
            pl.BlockSpec((2 * H, 8 * H), lambda k: (0, 0)),
            pl.BlockSpec((1, 8 * H), lambda k: (0, 0)),
        ],
        out_specs=pl.BlockSpec((TB, H), lambda k: (jnp.maximum(k - 1, 0), 0)),
        out_shape=jax.ShapeDtypeStruct((T, H), jnp.float32),
        scratch_shapes=[
            pltpu.VMEM((TB, 8 * H), jnp.float32),
            pltpu.VMEM((TB, H), jnp.float32),
            pltpu.VMEM((2, 2 * H), jnp.float32),
        ],
        compiler_params=pltpu.CompilerParams(
            dimension_semantics=("arbitrary",)),
        interpret=_INTERPRET,
    )(feat, wih_bd, whh_bd, bias)


def _prep_lstm_weights(W_ih, W_hh, b_ih, b_hh):
    wih_bd = jnp.zeros((2 * H, 8 * H), jnp.float32)
    whh_bd = jnp.zeros((2 * H, 8 * H), jnp.float32)
    bias = jnp.zeros((8 * H,), jnp.float32)
    for l in range(2):
        for gi in range(4):
            blk = 2 * gi + l
            cs = slice(blk * H, (blk + 1) * H)
            rs = slice(l * H, (l + 1) * H)
            gs = slice(gi * H, (gi + 1) * H)
            wih_bd = wih_bd.at[rs, cs].set(W_ih[l, gs, :].T)
            whh_bd = whh_bd.at[rs, cs].set(W_hh[l, gs, :].T)
            bias = bias.at[cs].set(b_ih[l, gs] + b_hh[l, gs])
    return wih_bd, whh_bd, bias[None, :]


def _bincounts(idx_stack):
    def one(ix):
        c = jnp.zeros((N,), jnp.float32).at[ix].add(1.0)
        return jnp.pad(c[:, None], ((0, 0), (0, 15)))
    return jax.vmap(one)(idx_stack)


def _edge_aggregate(q_stack, src_stack, dst_stack):
    def one(q, s, d):
        return jnp.zeros((N, H), jnp.float32).at[d].add(q[s])
    return jax.vmap(one)(q_stack, src_stack, dst_stack)


def kernel(x_node, x_instance, x_svc, ei_svc_call, ei_instance_node,
           ei_node_instance, ei_instance_instance, ei_svc_instance,
           ei_instance_svc, W_conv, b_conv, W_ih, W_hh, b_ih, b_hh):
    eis = [ei_svc_call, ei_instance_node, ei_node_instance,
           ei_instance_instance, ei_svc_instance, ei_instance_svc]
    idx_stack = jnp.stack([e[i] for e in eis for i in (0, 1)])
    counts = _bincounts(idx_stack)

    src_tables = jnp.stack(
        [x_svc, x_instance, x_node, x_instance, x_svc, x_instance])
    q_stack = _scaled_matmuls(src_tables, counts, W_conv)

    src_stack = jnp.stack([e[0] for e in eis])
    dst_stack = jnp.stack([e[1] for e in eis])
    agg = _edge_aggregate(q_stack, src_stack, dst_stack)

    zeros_nh = jnp.zeros((N, H), jnp.float32)
    agg_stack = jnp.stack([
        jnp.stack([agg[1], zeros_nh, zeros_nh]),
        jnp.stack([agg[2], agg[3], agg[4]]),
        jnp.stack([agg[0], agg[5], zeros_nh]),
    ])
    ones_cnt = jnp.ones((N, 16), jnp.float32)
    cnt_stack = jnp.stack([
        jnp.stack([counts[3], ones_cnt, ones_cnt]),
        jnp.stack([counts[5], counts[7], counts[9]]),
        jnp.stack([counts[1], counts[11], ones_cnt]),
    ])
    zeros_b = jnp.zeros((H,), jnp.float32)
    b_stack = jnp.stack([
        jnp.stack([b_conv[1], zeros_b, zeros_b]),
        jnp.stack([b_conv[2], b_conv[3], b_conv[4]]),
        jnp.stack([b_conv[0], b_conv[5], zeros_b]),
    ])

    feat3 = _combine(agg_stack, cnt_stack, b_stack)
    feat = feat3.reshape(T, D)

    wih_bd, whh_bd, bias = _prep_lstm_weights(W_ih, W_hh, b_ih, b_hh)
    rnn_out = _lstm(feat, wih_bd, whh_bd, bias)

    return (feat[:N], feat[N:2 * N], feat[2 * N:], rnn_out)


# speedup vs baseline: 15.4016x; 3.6569x over previous
import functools

import jax
import jax.numpy as jnp
from jax import lax
from jax.experimental import pallas as pl
from jax.experimental.pallas import tpu as pltpu
from jax.experimental.pallas import tpu_sc as plsc

N = 4000
E = 64000
D = 128
H = 128
T = 3 * N
TB = 600
NB = T // TB

NC = 2
NS = 16
EDGES_PER_TILE = E // NS
XTILES = 10
XROWS = N // XTILES
CH = 80

_INTERPRET = False


def _sc_mesh():
    return plsc.VectorSubcoreMesh(
        core_axis_name="c", subcore_axis_name="s",
        num_cores=NC, num_subcores=NS)


def _bincount_body(idx_hbm, ones_hbm, zeros_hbm, out_hbm,
                   idx_v, ones_v, t0, t1, t2, sem):
    c = lax.axis_index("c")
    s = lax.axis_index("s")
    tables = [t0, t1, t2]
    row0 = s * XROWS
    pltpu.sync_copy(ones_hbm, ones_v)
    for p in range(2):
        @pl.when(s < XTILES)
        def _zero():
            for j in range(3):
                pltpu.sync_copy(zeros_hbm, tables[j].at[pl.ds(row0, XROWS)])

        plsc.subcore_barrier()
        for j in range(3):
            base = (c * 6 + p * 3 + j) * E + s * EDGES_PER_TILE
            for ch in range(EDGES_PER_TILE // CH):
                pltpu.sync_copy(idx_hbm.at[pl.ds(base + ch * CH, CH)], idx_v)
                pltpu.sync_copy(ones_v, tables[j].at[idx_v], add=True)
        plsc.subcore_barrier()

        @pl.when(s < XTILES)
        def _export():
            for j in range(3):
                arr = c * 6 + p * 3 + j
                pltpu.sync_copy(
                    tables[j].at[pl.ds(row0, XROWS)],
                    out_hbm.at[pl.ds(arr * N + row0, XROWS)])

        plsc.subcore_barrier()


def _bincounts_sc(idx_flat):
    ones_rows = jnp.ones((CH, H), jnp.float32)
    zeros_rows = jnp.zeros((XROWS, H), jnp.float32)
    k = pl.kernel(
        _bincount_body,
        out_type=jax.ShapeDtypeStruct((12 * N, H), jnp.float32),
        mesh=_sc_mesh(),
        scratch_types=[
            pltpu.VMEM((CH,), jnp.int32),
            pltpu.VMEM((CH, H), jnp.float32),
        ] + [pltpu.VMEM_SHARED((N, H), jnp.float32)] * 3
          + [pltpu.SemaphoreType.DMA],
    )
    return k(idx_flat, ones_rows, zeros_rows).reshape(12, N, H)


def _aggregate_body(q_hbm, src_hbm, dst_hbm, zeros_hbm, out_hbm,
                    src_v, adj_v, dst_v, rows_v, a0, a1, a2, sem):
    c = lax.axis_index("c")
    s = lax.axis_index("s")
    accs = [a0, a1, a2]
    row0 = s * XROWS

    @pl.when(s < XTILES)
    def _zero():
        for j in range(3):
            pltpu.sync_copy(zeros_hbm, accs[j].at[pl.ds(row0, XROWS)])

    plsc.subcore_barrier()
    for j in range(3):
        r = c * 3 + j
        base = r * E + s * EDGES_PER_TILE
        for ch in range(EDGES_PER_TILE // CH):
            pltpu.sync_copy(src_hbm.at[pl.ds(base + ch * CH, CH)], src_v)
            pltpu.sync_copy(dst_hbm.at[pl.ds(base + ch * CH, CH)], dst_v)
            for v in range(CH // 16):
                adj_v[pl.ds(v * 16, 16)] = src_v[pl.ds(v * 16, 16)] + r * N
            pltpu.async_copy(q_hbm.at[adj_v], rows_v, sem).wait()
            pltpu.sync_copy(rows_v, accs[j].at[dst_v], add=True)
    plsc.subcore_barrier()

    @pl.when(s < XTILES)
    def _export():
        for j in range(3):
            r = c * 3 + j
            pltpu.sync_copy(
                accs[j].at[pl.ds(row0, XROWS)],
                out_hbm.at[pl.ds(r * N + row0, XROWS)])


def _edge_aggregate_sc(q_stack, src_flat, dst_flat):
    zeros_rows = jnp.zeros((XROWS, H), jnp.float32)
    k = pl.kernel(
        _aggregate_body,
        out_type=jax.ShapeDtypeStruct((6 * N, H), jnp.float32),
        mesh=_sc_mesh(),
        scratch_types=[
            pltpu.VMEM((CH,), jnp.int32),
            pltpu.VMEM((CH,), jnp.int32),
            pltpu.VMEM((CH,), jnp.int32),
            pltpu.VMEM((CH, H), jnp.float32),
        ] + [pltpu.VMEM_SHARED((N, H), jnp.float32)] * 3
          + [pltpu.SemaphoreType.DMA],
    )
    return k(q_stack.reshape(6 * N, H), src_flat, dst_flat,
             zeros_rows).reshape(6, N, H)


def _matmul_kernel(x_ref, cnt_ref, w_ref, q_ref):
    c = cnt_ref[0, :, 0]
    s = jax.lax.rsqrt(jnp.maximum(c, 1.0))
    x = x_ref[0] * s[:, None]
    q_ref[0] = jax.lax.dot_general(
        x, w_ref[0], (((1,), (0,)), ((), ())),
        preferred_element_type=jnp.float32)


def _scaled_matmuls(src_stack, counts, W_conv):
    return pl.pallas_call(
        _matmul_kernel,
        grid=(6,),
        in_specs=[
            pl.BlockSpec((1, N, D), lambda r: (r, 0, 0)),
            pl.BlockSpec((1, N, H), lambda r: (2 * r, 0, 0)),
            pl.BlockSpec((1, D, H), lambda r: (r, 0, 0)),
        ],
        out_specs=pl.BlockSpec((1, N, H), lambda r: (r, 0, 0)),
        out_shape=jax.ShapeDtypeStruct((6, N, H), jnp.float32),
        interpret=_INTERPRET,
    )(src_stack, counts, W_conv)


def _combine_kernel(agg_ref, cnt_ref, b_ref, feat_ref):
    t = pl.program_id(0)
    inv_m = jnp.where(t == 0, 1.0, jnp.where(t == 1, 1.0 / 3.0, 0.5))
    acc = jnp.zeros((N, H), jnp.float32)
    for j in range(3):
        c = cnt_ref[0, j, :, 0]
        s = jax.lax.rsqrt(jnp.maximum(c, 1.0))
        acc = acc + agg_ref[0, j] * s[:, None]
    beff = (b_ref[0, 0] + b_ref[0, 1] + b_ref[0, 2]) * inv_m
    feat_ref[0] = jnp.maximum(acc * inv_m + beff[None, :], 0.0)


def _combine(agg_stack, cnt_stack, b_stack):
    return pl.pallas_call(
        _combine_kernel,
        grid=(3,),
        in_specs=[
            pl.BlockSpec((1, 3, N, H), lambda t: (t, 0, 0, 0)),
            pl.BlockSpec((1, 3, N, H), lambda t: (t, 0, 0, 0)),
            pl.BlockSpec((1, 3, H), lambda t: (t, 0, 0)),
        ],
        out_specs=pl.BlockSpec((1, N, H), lambda t: (t, 0, 0)),
        out_shape=jax.ShapeDtypeStruct((3, N, H), jnp.float32),
        interpret=_INTERPRET,
    )(agg_stack, cnt_stack, b_stack)


def _lstm_kernel(feat_ref, wihbd_ref, whhbd_ref, b_ref, out_ref,
                 xp_scr, h1blk, carry):
    k = pl.program_id(0)

    @pl.when(k == 0)
    def _init():
        carry[...] = jnp.zeros_like(carry)
        h1blk[...] = jnp.zeros_like(h1blk)

    @pl.when(k == 1)
    def _reset_l2():
        carry[:, H:2 * H] = jnp.zeros((2, H), jnp.float32)

    mm = functools.partial(
        jax.lax.dot_general, dimension_numbers=(((1,), (0,)), ((), ())),
        preferred_element_type=jnp.float32)

    xin = jnp.concatenate([feat_ref[...], h1blk[...]], axis=1)
    xp_scr[...] = mm(xin, wihbd_ref[...]) + b_ref[...]

    whh = whhbd_ref[...]
    h12 = carry[0:1, :]
    c12 = carry[1:2, :]

    def group(gi, hc):
        h12, c12 = hc
        base = pl.multiple_of(gi * 8, 8)
        rows = xp_scr[pl.ds(base, 8), :]
        h1s, outs = [], []
        for j in range(8):
            g = rows[j:j + 1, :] + mm(h12, whh)
            ii = jax.nn.sigmoid(g[:, 0:2 * H])
            ff = jax.nn.sigmoid(g[:, 2 * H:4 * H])
            gg = jnp.tanh(g[:, 4 * H:6 * H])
            oo = jax.nn.sigmoid(g[:, 6 * H:8 * H])
            c12 = ff * c12 + ii * gg
            h12 = oo * jnp.tanh(c12)
            h1s.append(h12[:, 0:H])
            outs.append(h12[:, H:2 * H])
        h1blk[pl.ds(base, 8), :] = jnp.concatenate(h1s, axis=0)
        out_ref[pl.ds(base, 8), :] = jnp.maximum(
            jnp.concatenate(outs, axis=0), 0.0)
        return (h12, c12)

    h12, c12 = jax.lax.fori_loop(0, TB // 8, group, (h12, c12))
    carry[0:1, :] = h12
    carry[1:2, :] = c12


def _lstm(feat, wih_bd, whh_bd, bias):
    return pl.pallas_call(
        _lstm_kernel,
        grid=(NB + 1,),
        in_specs=[
            pl.BlockSpec((TB, D), lambda k: (jnp.minimum(k, NB - 1), 0)),
            pl.BlockSpec((2 * H, 8 * H), lambda k: (0, 0)),
            pl.BlockSpec((2 * H, 8 * H), lambda k: (0, 0)),
            pl.BlockSpec((1, 8 * H), lambda k: (0, 0)),
        ],
        out_specs=pl.BlockSpec((TB, H), lambda k: (jnp.maximum(k - 1, 0), 0)),
        out_shape=jax.ShapeDtypeStruct((T, H), jnp.float32),
        scratch_shapes=[
            pltpu.VMEM((TB, 8 * H), jnp.float32),
            pltpu.VMEM((TB, H), jnp.float32),
            pltpu.VMEM((2, 2 * H), jnp.float32),
        ],
        compiler_params=pltpu.CompilerParams(
            dimension_semantics=("arbitrary",)),
        interpret=_INTERPRET,
    )(feat, wih_bd, whh_bd, bias)


def _prep_lstm_weights(W_ih, W_hh, b_ih, b_hh):
    wih_bd = jnp.zeros((2 * H, 8 * H), jnp.float32)
    whh_bd = jnp.zeros((2 * H, 8 * H), jnp.float32)
    bias = jnp.zeros((8 * H,), jnp.float32)
    for l in range(2):
        for gi in range(4):
            blk = 2 * gi + l
            cs = slice(blk * H, (blk + 1) * H)
            rs = slice(l * H, (l + 1) * H)
            gs = slice(gi * H, (gi + 1) * H)
            wih_bd = wih_bd.at[rs, cs].set(W_ih[l, gs, :].T)
            whh_bd = whh_bd.at[rs, cs].set(W_hh[l, gs, :].T)
            bias = bias.at[cs].set(b_ih[l, gs] + b_hh[l, gs])
    return wih_bd, whh_bd, bias[None, :]


def kernel(x_node, x_instance, x_svc, ei_svc_call, ei_instance_node,
           ei_node_instance, ei_instance_instance, ei_svc_instance,
           ei_instance_svc, W_conv, b_conv, W_ih, W_hh, b_ih, b_hh):
    eis = [ei_svc_call, ei_instance_node, ei_node_instance,
           ei_instance_instance, ei_svc_instance, ei_instance_svc]
    idx_stack = jnp.stack([e[i] for e in eis for i in (0, 1)])
    counts = _bincounts_sc(idx_stack.reshape(-1))

    src_tables = jnp.stack(
        [x_svc, x_instance, x_node, x_instance, x_svc, x_instance])
    q_stack = _scaled_matmuls(src_tables, counts, W_conv)

    src_flat = jnp.stack([e[0] for e in eis]).reshape(-1)
    dst_flat = jnp.stack([e[1] for e in eis]).reshape(-1)
    agg = _edge_aggregate_sc(q_stack, src_flat, dst_flat)

    zeros_nh = jnp.zeros((N, H), jnp.float32)
    agg_stack = jnp.stack([
        jnp.stack([agg[1], zeros_nh, zeros_nh]),
        jnp.stack([agg[2], agg[3], agg[4]]),
        jnp.stack([agg[0], agg[5], zeros_nh]),
    ])
    ones_cnt = jnp.ones((N, H), jnp.float32)
    cnt_stack = jnp.stack([
        jnp.stack([counts[3], ones_cnt, ones_cnt]),
        jnp.stack([counts[5], counts[7], counts[9]]),
        jnp.stack([counts[1], counts[11], ones_cnt]),
    ])
    zeros_b = jnp.zeros((H,), jnp.float32)
    b_stack = jnp.stack([
        jnp.stack([b_conv[1], zeros_b, zeros_b]),
        jnp.stack([b_conv[2], b_conv[3], b_conv[4]]),
        jnp.stack([b_conv[0], b_conv[5], zeros_b]),
    ])

    feat3 = _combine(agg_stack, cnt_stack, b_stack)
    feat = feat3.reshape(T, D)

    wih_bd, whh_bd, bias = _prep_lstm_weights(W_ih, W_hh, b_ih, b_hh)
    rnn_out = _lstm(feat, wih_bd, whh_bd, bias)

    return (feat[:N], feat[N:2 * N], feat[2 * N:], rnn_out)


# speedup vs baseline: 16.2669x; 1.0562x over previous
import functools

import jax
import jax.numpy as jnp
from jax import lax
from jax.experimental import pallas as pl
from jax.experimental.pallas import tpu as pltpu
from jax.experimental.pallas import tpu_sc as plsc

N = 4000
E = 64000
D = 128
H = 128
T = 3 * N
TB = 600
NB = T // TB

NC = 2
NS = 16
EDGES_PER_TILE = E // NS
XTILES = 10
XROWS = N // XTILES
CH = 80

_INTERPRET = False


def _sc_mesh():
    return plsc.VectorSubcoreMesh(
        core_axis_name="c", subcore_axis_name="s",
        num_cores=NC, num_subcores=NS)


def _bincount_body(idx_hbm, ones_hbm, zeros_hbm, out_hbm,
                   idx_a, idx_b, ones_v, t0, t1, t2, sem_a, sem_b):
    c = lax.axis_index("c")
    s = lax.axis_index("s")
    tables = [t0, t1, t2]
    row0 = s * XROWS
    pltpu.sync_copy(ones_hbm, ones_v)
    for p in range(2):
        @pl.when(s < XTILES)
        def _zero():
            for j in range(3):
                pltpu.sync_copy(zeros_hbm, tables[j].at[pl.ds(row0, XROWS)])

        plsc.subcore_barrier()
        for j in range(3):
            base = (c * 6 + p * 3 + j) * E + s * EDGES_PER_TILE

            def _pair(i, _):
                off = base + i * (2 * CH)
                pltpu.sync_copy(idx_hbm.at[pl.ds(off, CH)], idx_a)
                sa = pltpu.async_copy(
                    ones_v, tables[j].at[idx_a], sem_a, add=True)
                pltpu.sync_copy(idx_hbm.at[pl.ds(off + CH, CH)], idx_b)
                sb = pltpu.async_copy(
                    ones_v, tables[j].at[idx_b], sem_b, add=True)
                sa.wait()
                sb.wait()
                return 0

            lax.fori_loop(0, EDGES_PER_TILE // (2 * CH), _pair, 0)
        plsc.subcore_barrier()

        @pl.when(s < XTILES)
        def _export():
            for j in range(3):
                arr = c * 6 + p * 3 + j
                pltpu.sync_copy(
                    tables[j].at[pl.ds(row0, XROWS)],
                    out_hbm.at[pl.ds(arr * N + row0, XROWS)])

        plsc.subcore_barrier()


def _bincounts_sc(idx_flat):
    ones_rows = jnp.ones((CH, H), jnp.float32)
    zeros_rows = jnp.zeros((XROWS, H), jnp.float32)
    k = pl.kernel(
        _bincount_body,
        out_type=jax.ShapeDtypeStruct((12 * N, H), jnp.float32),
        mesh=_sc_mesh(),
        scratch_types=[
            pltpu.VMEM((CH,), jnp.int32),
            pltpu.VMEM((CH,), jnp.int32),
            pltpu.VMEM((CH, H), jnp.float32),
        ] + [pltpu.VMEM_SHARED((N, H), jnp.float32)] * 3
          + [pltpu.SemaphoreType.DMA] * 2,
    )
    return k(idx_flat, ones_rows, zeros_rows).reshape(12, N, H)


def _aggregate_body(q_hbm, src_hbm, dst_hbm, zeros_hbm, out_hbm,
                    src_a, adj_a, dst_a, rows_a, src_b, adj_b, dst_b, rows_b,
                    a0, a1, a2, sem_ga, sem_gb, sem_sa, sem_sb):
    c = lax.axis_index("c")
    s = lax.axis_index("s")
    accs = [a0, a1, a2]
    row0 = s * XROWS

    @pl.when(s < XTILES)
    def _zero():
        for j in range(3):
            pltpu.sync_copy(zeros_hbm, accs[j].at[pl.ds(row0, XROWS)])

    plsc.subcore_barrier()
    for j in range(3):
        r = c * 3 + j
        base = r * E + s * EDGES_PER_TILE

        def _pair(i, _):
            off = base + i * (2 * CH)
            pltpu.sync_copy(src_hbm.at[pl.ds(off, CH)], src_a)
            pltpu.sync_copy(dst_hbm.at[pl.ds(off, CH)], dst_a)
            for v in range(CH // 16):
                adj_a[pl.ds(v * 16, 16)] = src_a[pl.ds(v * 16, 16)] + r * N
            ga = pltpu.async_copy(q_hbm.at[adj_a], rows_a, sem_ga)
            pltpu.sync_copy(src_hbm.at[pl.ds(off + CH, CH)], src_b)
            pltpu.sync_copy(dst_hbm.at[pl.ds(off + CH, CH)], dst_b)
            for v in range(CH // 16):
                adj_b[pl.ds(v * 16, 16)] = src_b[pl.ds(v * 16, 16)] + r * N
            gb = pltpu.async_copy(q_hbm.at[adj_b], rows_b, sem_gb)
            ga.wait()
            sa = pltpu.async_copy(rows_a, accs[j].at[dst_a], sem_sa, add=True)
            gb.wait()
            sb = pltpu.async_copy(rows_b, accs[j].at[dst_b], sem_sb, add=True)
            sa.wait()
            sb.wait()
            return 0

        lax.fori_loop(0, EDGES_PER_TILE // (2 * CH), _pair, 0)
    plsc.subcore_barrier()

    @pl.when(s < XTILES)
    def _export():
        for j in range(3):
            r = c * 3 + j
            pltpu.sync_copy(
                accs[j].at[pl.ds(row0, XROWS)],
                out_hbm.at[pl.ds(r * N + row0, XROWS)])


def _edge_aggregate_sc(q_stack, src_flat, dst_flat):
    zeros_rows = jnp.zeros((XROWS, H), jnp.float32)
    k = pl.kernel(
        _aggregate_body,
        out_type=jax.ShapeDtypeStruct((6 * N, H), jnp.float32),
        mesh=_sc_mesh(),
        scratch_types=[
            pltpu.VMEM((CH,), jnp.int32),
            pltpu.VMEM((CH,), jnp.int32),
            pltpu.VMEM((CH,), jnp.int32),
            pltpu.VMEM((CH, H), jnp.float32),
            pltpu.VMEM((CH,), jnp.int32),
            pltpu.VMEM((CH,), jnp.int32),
            pltpu.VMEM((CH,), jnp.int32),
            pltpu.VMEM((CH, H), jnp.float32),
        ] + [pltpu.VMEM_SHARED((N, H), jnp.float32)] * 3
          + [pltpu.SemaphoreType.DMA] * 4,
    )
    return k(q_stack.reshape(6 * N, H), src_flat, dst_flat,
             zeros_rows).reshape(6, N, H)


def _matmul_kernel(x_ref, cnt_ref, w_ref, q_ref):
    c = cnt_ref[0, :, 0]
    s = jax.lax.rsqrt(jnp.maximum(c, 1.0))
    x = x_ref[0] * s[:, None]
    q_ref[0] = jax.lax.dot_general(
        x, w_ref[0], (((1,), (0,)), ((), ())),
        preferred_element_type=jnp.float32)


def _scaled_matmuls(src_stack, counts, W_conv):
    return pl.pallas_call(
        _matmul_kernel,
        grid=(6,),
        in_specs=[
            pl.BlockSpec((1, N, D), lambda r: (r, 0, 0)),
            pl.BlockSpec((1, N, H), lambda r: (2 * r, 0, 0)),
            pl.BlockSpec((1, D, H), lambda r: (r, 0, 0)),
        ],
        out_specs=pl.BlockSpec((1, N, H), lambda r: (r, 0, 0)),
        out_shape=jax.ShapeDtypeStruct((6, N, H), jnp.float32),
        interpret=_INTERPRET,
    )(src_stack, counts, W_conv)


def _combine_kernel(agg_ref, cnt_ref, b_ref, feat_ref):
    t = pl.program_id(0)
    inv_m = jnp.where(t == 0, 1.0, jnp.where(t == 1, 1.0 / 3.0, 0.5))
    acc = jnp.zeros((N, H), jnp.float32)
    for j in range(3):
        c = cnt_ref[0, j, :, 0]
        s = jax.lax.rsqrt(jnp.maximum(c, 1.0))
        acc = acc + agg_ref[0, j] * s[:, None]
    beff = (b_ref[0, 0] + b_ref[0, 1] + b_ref[0, 2]) * inv_m
    feat_ref[0] = jnp.maximum(acc * inv_m + beff[None, :], 0.0)


def _combine(agg_stack, cnt_stack, b_stack):
    return pl.pallas_call(
        _combine_kernel,
        grid=(3,),
        in_specs=[
            pl.BlockSpec((1, 3, N, H), lambda t: (t, 0, 0, 0)),
            pl.BlockSpec((1, 3, N, H), lambda t: (t, 0, 0, 0)),
            pl.BlockSpec((1, 3, H), lambda t: (t, 0, 0)),
        ],
        out_specs=pl.BlockSpec((1, N, H), lambda t: (t, 0, 0)),
        out_shape=jax.ShapeDtypeStruct((3, N, H), jnp.float32),
        interpret=_INTERPRET,
    )(agg_stack, cnt_stack, b_stack)


def _lstm_kernel(feat_ref, wihbd_ref, whhbd_ref, b_ref, out_ref,
                 xp_scr, h1blk, carry):
    k = pl.program_id(0)

    @pl.when(k == 0)
    def _init():
        carry[...] = jnp.zeros_like(carry)
        h1blk[...] = jnp.zeros_like(h1blk)

    @pl.when(k == 1)
    def _reset_l2():
        carry[:, H:2 * H] = jnp.zeros((2, H), jnp.float32)

    mm = functools.partial(
        jax.lax.dot_general, dimension_numbers=(((1,), (0,)), ((), ())),
        preferred_element_type=jnp.float32)

    xin = jnp.concatenate([feat_ref[...], h1blk[...]], axis=1)
    xp_scr[...] = mm(xin, wihbd_ref[...]) + b_ref[...]

    whh = whhbd_ref[...].astype(jnp.bfloat16)
    h12 = carry[0:1, :]
    c12 = carry[1:2, :]

    def group(gi, hc):
        h12, c12 = hc
        base = pl.multiple_of(gi * 8, 8)
        rows = xp_scr[pl.ds(base, 8), :]
        h1s, outs = [], []
        for j in range(8):
            g = rows[j:j + 1, :] + mm(h12.astype(jnp.bfloat16), whh)
            ii = jax.nn.sigmoid(g[:, 0:2 * H])
            ff = jax.nn.sigmoid(g[:, 2 * H:4 * H])
            gg = jnp.tanh(g[:, 4 * H:6 * H])
            oo = jax.nn.sigmoid(g[:, 6 * H:8 * H])
            c12 = ff * c12 + ii * gg
            h12 = oo * jnp.tanh(c12)
            h1s.append(h12[:, 0:H])
            outs.append(h12[:, H:2 * H])
        h1blk[pl.ds(base, 8), :] = jnp.concatenate(h1s, axis=0)
        out_ref[pl.ds(base, 8), :] = jnp.maximum(
            jnp.concatenate(outs, axis=0), 0.0)
        return (h12, c12)

    h12, c12 = jax.lax.fori_loop(0, TB // 8, group, (h12, c12))
    carry[0:1, :] = h12
    carry[1:2, :] = c12


def _lstm(feat, wih_bd, whh_bd, bias):
    return pl.pallas_call(
        _lstm_kernel,
        grid=(NB + 1,),
        in_specs=[
            pl.BlockSpec((TB, D), lambda k: (jnp.minimum(k, NB - 1), 0)),
            pl.BlockSpec((2 * H, 8 * H), lambda k: (0, 0)),
            pl.BlockSpec((2 * H, 8 * H), lambda k: (0, 0)),
            pl.BlockSpec((1, 8 * H), lambda k: (0, 0)),
        ],
        out_specs=pl.BlockSpec((TB, H), lambda k: (jnp.maximum(k - 1, 0), 0)),
        out_shape=jax.ShapeDtypeStruct((T, H), jnp.float32),
        scratch_shapes=[
            pltpu.VMEM((TB, 8 * H), jnp.float32),
            pltpu.VMEM((TB, H), jnp.float32),
            pltpu.VMEM((2, 2 * H), jnp.float32),
        ],
        compiler_params=pltpu.CompilerParams(
            dimension_semantics=("arbitrary",)),
        interpret=_INTERPRET,
    )(feat, wih_bd, whh_bd, bias)


def _prep_lstm_weights(W_ih, W_hh, b_ih, b_hh):
    wih_bd = jnp.zeros((2 * H, 8 * H), jnp.float32)
    whh_bd = jnp.zeros((2 * H, 8 * H), jnp.float32)
    bias = jnp.zeros((8 * H,), jnp.float32)
    for l in range(2):
        for gi in range(4):
            blk = 2 * gi + l
            cs = slice(blk * H, (blk + 1) * H)
            rs = slice(l * H, (l + 1) * H)
            gs = slice(gi * H, (gi + 1) * H)
            wih_bd = wih_bd.at[rs, cs].set(W_ih[l, gs, :].T)
            whh_bd = whh_bd.at[rs, cs].set(W_hh[l, gs, :].T)
            bias = bias.at[cs].set(b_ih[l, gs] + b_hh[l, gs])
    return wih_bd, whh_bd, bias[None, :]


def kernel(x_node, x_instance, x_svc, ei_svc_call, ei_instance_node,
           ei_node_instance, ei_instance_instance, ei_svc_instance,
           ei_instance_svc, W_conv, b_conv, W_ih, W_hh, b_ih, b_hh):
    eis = [ei_svc_call, ei_instance_node, ei_node_instance,
           ei_instance_instance, ei_svc_instance, ei_instance_svc]
    idx_stack = jnp.stack([e[i] for e in eis for i in (0, 1)])
    counts = _bincounts_sc(idx_stack.reshape(-1))

    src_tables = jnp.stack(
        [x_svc, x_instance, x_node, x_instance, x_svc, x_instance])
    q_stack = _scaled_matmuls(src_tables, counts, W_conv)

    src_flat = jnp.stack([e[0] for e in eis]).reshape(-1)
    dst_flat = jnp.stack([e[1] for e in eis]).reshape(-1)
    agg = _edge_aggregate_sc(q_stack, src_flat, dst_flat)

    zeros_nh = jnp.zeros((N, H), jnp.float32)
    agg_stack = jnp.stack([
        jnp.stack([agg[1], zeros_nh, zeros_nh]),
        jnp.stack([agg[2], agg[3], agg[4]]),
        jnp.stack([agg[0], agg[5], zeros_nh]),
    ])
    ones_cnt = jnp.ones((N, H), jnp.float32)
    cnt_stack = jnp.stack([
        jnp.stack([counts[3], ones_cnt, ones_cnt]),
        jnp.stack([counts[5], counts[7], counts[9]]),
        jnp.stack([counts[1], counts[11], ones_cnt]),
    ])
    zeros_b = jnp.zeros((H,), jnp.float32)
    b_stack = jnp.stack([
        jnp.stack([b_conv[1], zeros_b, zeros_b]),
        jnp.stack([b_conv[2], b_conv[3], b_conv[4]]),
        jnp.stack([b_conv[0], b_conv[5], zeros_b]),
    ])

    feat3 = _combine(agg_stack, cnt_stack, b_stack)
    feat = feat3.reshape(T, D)

    wih_bd, whh_bd, bias = _prep_lstm_weights(W_ih, W_hh, b_ih, b_hh)
    rnn_out = _lstm(feat, wih_bd, whh_bd, bias)

    return (feat[:N], feat[N:2 * N], feat[2 * N:], rnn_out)
